# Initial kernel scaffold; baseline (speedup 1.0000x reference)
import jax
import jax.numpy as jnp
from jax.experimental import pallas as pl


def kernel(gen_logits):
    raise NotImplementedError("write your pallas kernel here")


# speedup vs baseline: 1.1900x; 1.1900x over previous
import jax
import jax.numpy as jnp
from jax.experimental import pallas as pl

SEQ = 32
BATCH = 32
VOCAB = 100000
ROWS = SEQ * BATCH
BLOCK_ROWS = 8
GRID = ROWS // BLOCK_ROWS


def _row_body(x_ref, g_ref, ids_ref, logp_ref):
    x = x_ref[...]
    g = g_ref[...]
    pert = x + g
    ids = jnp.argmax(pert, axis=-1).astype(jnp.int32)
    m = jnp.max(x, axis=-1)
    s = jnp.sum(jnp.exp(x - m[:, None]), axis=-1)
    lse = m + jnp.log(s)
    col = jax.lax.broadcasted_iota(jnp.int32, x.shape, 1)
    xat = jnp.sum(jnp.where(col == ids[:, None], x, 0.0), axis=-1)
    ids_ref[...] = ids.reshape(1, 1, BLOCK_ROWS)
    logp_ref[...] = (xat - lse).reshape(1, 1, BLOCK_ROWS)


def kernel(gen_logits):
    gkey = jax.random.key(42)
    gumbel = jax.random.gumbel(gkey, gen_logits.shape, dtype=gen_logits.dtype)
    x2 = gen_logits.reshape(ROWS, VOCAB)
    g2 = gumbel.reshape(ROWS, VOCAB)

    ids3, logp3 = pl.pallas_call(
        _row_body,
        grid=(GRID,),
        in_specs=[
            pl.BlockSpec((BLOCK_ROWS, VOCAB), lambda i: (i, 0)),
            pl.BlockSpec((BLOCK_ROWS, VOCAB), lambda i: (i, 0)),
        ],
        out_specs=[
            pl.BlockSpec((1, 1, BLOCK_ROWS), lambda i: (i, 0, 0)),
            pl.BlockSpec((1, 1, BLOCK_ROWS), lambda i: (i, 0, 0)),
        ],
        out_shape=[
            jax.ShapeDtypeStruct((GRID, 1, BLOCK_ROWS), jnp.int32),
            jax.ShapeDtypeStruct((GRID, 1, BLOCK_ROWS), jnp.float32),
        ],
    )(x2, g2)

    ids = ids3.reshape(SEQ, BATCH)
    logp = logp3.reshape(SEQ, BATCH)
    generated_tensor = ids.T.astype(jnp.int64)
    return (generated_tensor, logp.T)
